# baseline (speedup 1.0000x reference)
import functools

import jax
import jax.numpy as jnp
from jax.experimental import pallas as pl
from jax.experimental.pallas import tpu as pltpu


def _lstm_body(
    x_ref,
    wcat_ref,
    bias_ref,
    w2_ref,
    b2_ref,
    out_ref,
    h_scr,
):
    L, Bb, Dinp = x_ref.shape
    Hp = h_scr.shape[-1]
    Op = out_ref.shape[-1]

    wcat = wcat_ref[...]
    bias = bias_ref[...]

    def step(t, carry):
        h_prev, c_prev = carry
        zt = jnp.concatenate([x_ref[t], h_prev], axis=1)
        gates = bias + jnp.dot(zt, wcat,
                               preferred_element_type=jnp.float32)
        ti = jnp.tanh(gates[:, 0 * Hp:1 * Hp])
        tf = jnp.tanh(gates[:, 1 * Hp:2 * Hp])
        tg = jnp.tanh(gates[:, 2 * Hp:3 * Hp])
        to = jnp.tanh(gates[:, 3 * Hp:4 * Hp])
        i_g = 0.5 + 0.5 * ti
        f_g = 0.5 + 0.5 * tf
        o_g = 0.5 + 0.5 * to
        c_new = f_g * c_prev + i_g * tg
        h_new = (o_g * jnp.tanh(c_new)).astype(h_scr.dtype)
        h_scr[t] = h_new
        return h_new, c_new

    h0 = jnp.zeros((Bb, Hp), dtype=jnp.bfloat16)
    c0 = jnp.zeros((Bb, Hp), dtype=jnp.float32)
    jax.lax.fori_loop(0, L, step, (h0, c0), unroll=4)

    h2 = h_scr[...].reshape(L * Bb, Hp)
    y = jnp.dot(h2, w2_ref[...], preferred_element_type=jnp.float32) \
        + b2_ref[...]
    out_ref[...] = y.reshape(L, Bb, Op)


def _round_up(n, m):
    return ((n + m - 1) // m) * m


def _pad_to(a, shape):
    if tuple(a.shape) == tuple(shape):
        return a
    return jnp.pad(a, [(0, s - d) for d, s in zip(a.shape, shape)])


def _gate_scale_cols(a, H):
    scale = jnp.concatenate([
        jnp.full((H,), 0.5, a.dtype), jnp.full((H,), 0.5, a.dtype),
        jnp.ones((H,), a.dtype), jnp.full((H,), 0.5, a.dtype)
    ])
    return a * scale


def _pad_gate_cols(a, H, Hp):
    if H == Hp:
        return a
    parts = []
    for k in range(4):
        blk = a[..., k * H:(k + 1) * H]
        parts.append(jnp.pad(blk, [(0, 0)] * (a.ndim - 1) + [(0, Hp - H)]))
    return jnp.concatenate(parts, axis=-1)


@functools.partial(jax.jit, static_argnames=("hidden_dim", "output_dim"))
def _forward(x, params, hidden_dim, output_dim):
    B, L, Din = x.shape
    H, O = hidden_dim, output_dim
    f32, bf16 = jnp.float32, jnp.bfloat16

    Bp = _round_up(B, 8)
    Dinp = _round_up(Din, 128)
    Hp = _round_up(H, 128)
    Op = _round_up(O, 128)

    wf = params["w1"] @ params["wih"]
    bias = params["b1"] @ params["wih"] + params["bih"] + params["bhh"]

    wf_p = _pad_gate_cols(_pad_to(wf, (Dinp, 4 * H)), H, Hp)
    whh_p = _pad_gate_cols(_pad_to(params["whh"], (Hp, 4 * H)), H, Hp)
    wcat_p = _gate_scale_cols(
        jnp.concatenate([wf_p, whh_p], axis=0), Hp).astype(bf16)
    bias_p = _gate_scale_cols(_pad_gate_cols(bias, H, Hp), Hp).astype(f32)
    w2_p = _pad_to(params["w2"], (Hp, Op)).astype(bf16)
    b2_p = _pad_to(params["b2"], (1, Op)).astype(f32)

    x_tm = jnp.transpose(x, (1, 0, 2))
    x_p = _pad_to(x_tm, (L, Bp, Dinp)).astype(bf16)

    out_p = pl.pallas_call(
        _lstm_body,
        out_shape=jax.ShapeDtypeStruct((L, Bp, Op), jnp.float32),
        grid_spec=pltpu.PrefetchScalarGridSpec(
            num_scalar_prefetch=0,
            grid=(1,),
            in_specs=[
                pl.BlockSpec((L, Bp, Dinp), lambda i: (0, 0, 0)),
                pl.BlockSpec((Dinp + Hp, 4 * Hp), lambda i: (0, 0)),
                pl.BlockSpec((1, 4 * Hp), lambda i: (0, 0)),
                pl.BlockSpec((Hp, Op), lambda i: (0, 0)),
                pl.BlockSpec((1, Op), lambda i: (0, 0)),
            ],
            out_specs=pl.BlockSpec((L, Bp, Op), lambda i: (0, 0, 0)),
            scratch_shapes=[
                pltpu.VMEM((L, Bp, Hp), jnp.bfloat16),
            ],
        ),
        compiler_params=pltpu.CompilerParams(
            dimension_semantics=("arbitrary",),
            vmem_limit_bytes=100 * 1024 * 1024),
    )(x_p, wcat_p, bias_p, w2_p, b2_p)

    return jnp.transpose(out_p[:, :B, :O], (1, 0, 2))


def kernel(x, w1, b1, wih, whh, bih, bhh, w2, b2):
    params = {
        "w1": w1, "b1": b1,
        "wih": wih, "whh": whh, "bih": bih, "bhh": bhh,
        "w2": w2, "b2": b2,
    }
    return _forward(x, params, hidden_dim=512, output_dim=256)


# speedup vs baseline: 1.1293x; 1.1293x over previous
import functools

import jax
import jax.numpy as jnp
from jax.experimental import pallas as pl
from jax.experimental.pallas import tpu as pltpu


def _lstm_body(
    x_ref,
    wf_ref,
    bias_ref,
    whh_ref,
    w2_ref,
    b2_ref,
    out_ref,
    gx_scr,
    h_scr,
    hc_scr,
    c_scr,
):
    TL, Bb, Dinp = x_ref.shape
    Hp = h_scr.shape[-1]
    Op = out_ref.shape[-1]

    t_id = pl.program_id(0)

    @pl.when(t_id == 0)
    def _():
        hc_scr[...] = jnp.zeros((Bb, Hp), dtype=hc_scr.dtype)
        c_scr[...] = jnp.zeros((Bb, Hp), dtype=c_scr.dtype)

    x2 = x_ref[...].reshape(TL * Bb, Dinp)
    gx = jnp.dot(x2, wf_ref[...], preferred_element_type=jnp.float32) \
        + bias_ref[...]
    gx_scr[...] = gx.reshape(TL, Bb, 4 * Hp).astype(gx_scr.dtype)

    whh = whh_ref[...]

    def step(t, carry):
        h_prev, c_prev = carry
        gates = gx_scr[t].astype(jnp.float32) + jnp.dot(
            h_prev, whh, preferred_element_type=jnp.float32)
        ti = jnp.tanh(gates[:, 0 * Hp:1 * Hp])
        tf = jnp.tanh(gates[:, 1 * Hp:2 * Hp])
        tg = jnp.tanh(gates[:, 2 * Hp:3 * Hp])
        to = jnp.tanh(gates[:, 3 * Hp:4 * Hp])
        i_g = 0.5 + 0.5 * ti
        f_g = 0.5 + 0.5 * tf
        o_g = 0.5 + 0.5 * to
        c_new = f_g * c_prev + i_g * tg
        h_new = (o_g * jnp.tanh(c_new)).astype(h_scr.dtype)
        h_scr[t] = h_new
        return h_new, c_new

    h0 = hc_scr[...]
    c0 = c_scr[...]
    hN, cN = jax.lax.fori_loop(0, TL, step, (h0, c0), unroll=4)
    hc_scr[...] = hN
    c_scr[...] = cN

    h2 = h_scr[...].reshape(TL * Bb, Hp)
    y = jnp.dot(h2, w2_ref[...], preferred_element_type=jnp.float32) \
        + b2_ref[...]
    out_ref[...] = y.reshape(TL, Bb, Op)


def _round_up(n, m):
    return ((n + m - 1) // m) * m


def _pad_to(a, shape):
    if tuple(a.shape) == tuple(shape):
        return a
    return jnp.pad(a, [(0, s - d) for d, s in zip(a.shape, shape)])


def _gate_scale_cols(a, H):
    scale = jnp.concatenate([
        jnp.full((H,), 0.5, a.dtype), jnp.full((H,), 0.5, a.dtype),
        jnp.ones((H,), a.dtype), jnp.full((H,), 0.5, a.dtype)
    ])
    return a * scale


def _pad_gate_cols(a, H, Hp):
    if H == Hp:
        return a
    parts = []
    for k in range(4):
        blk = a[..., k * H:(k + 1) * H]
        parts.append(jnp.pad(blk, [(0, 0)] * (a.ndim - 1) + [(0, Hp - H)]))
    return jnp.concatenate(parts, axis=-1)


@functools.partial(jax.jit, static_argnames=("hidden_dim", "output_dim"))
def _forward(x, params, hidden_dim, output_dim):
    B, L, Din = x.shape
    H, O = hidden_dim, output_dim
    f32, bf16 = jnp.float32, jnp.bfloat16

    Bp = _round_up(B, 8)
    Dinp = _round_up(Din, 128)
    Hp = _round_up(H, 128)
    Op = _round_up(O, 128)

    TL = min(64, L)
    Lp = _round_up(L, TL)

    wf = _gate_scale_cols(params["w1"] @ params["wih"], H)
    bias = _gate_scale_cols(
        params["b1"] @ params["wih"] + params["bih"] + params["bhh"], H)
    whh = _gate_scale_cols(params["whh"], H)

    wf_p = _pad_gate_cols(_pad_to(wf, (Dinp, 4 * H)), H, Hp).astype(bf16)
    bias_p = _pad_gate_cols(bias, H, Hp).astype(f32)
    whh_p = _pad_gate_cols(_pad_to(whh, (Hp, 4 * H)), H, Hp).astype(bf16)
    w2_p = _pad_to(params["w2"], (Hp, Op)).astype(bf16)
    b2_p = _pad_to(params["b2"], (1, Op)).astype(f32)

    x_tm = jnp.transpose(x, (1, 0, 2))
    x_p = _pad_to(x_tm, (Lp, Bp, Dinp)).astype(bf16)

    out_p = pl.pallas_call(
        _lstm_body,
        out_shape=jax.ShapeDtypeStruct((Lp, Bp, Op), jnp.float32),
        grid_spec=pltpu.PrefetchScalarGridSpec(
            num_scalar_prefetch=0,
            grid=(Lp // TL,),
            in_specs=[
                pl.BlockSpec((TL, Bp, Dinp), lambda l: (l, 0, 0)),
                pl.BlockSpec((Dinp, 4 * Hp), lambda l: (0, 0),
                             pipeline_mode=pl.Buffered(1)),
                pl.BlockSpec((1, 4 * Hp), lambda l: (0, 0),
                             pipeline_mode=pl.Buffered(1)),
                pl.BlockSpec((Hp, 4 * Hp), lambda l: (0, 0),
                             pipeline_mode=pl.Buffered(1)),
                pl.BlockSpec((Hp, Op), lambda l: (0, 0),
                             pipeline_mode=pl.Buffered(1)),
                pl.BlockSpec((1, Op), lambda l: (0, 0),
                             pipeline_mode=pl.Buffered(1)),
            ],
            out_specs=pl.BlockSpec((TL, Bp, Op), lambda l: (l, 0, 0)),
            scratch_shapes=[
                pltpu.VMEM((TL, Bp, 4 * Hp), jnp.bfloat16),
                pltpu.VMEM((TL, Bp, Hp), jnp.bfloat16),
                pltpu.VMEM((Bp, Hp), jnp.bfloat16),
                pltpu.VMEM((Bp, Hp), jnp.float32),
            ],
        ),
        compiler_params=pltpu.CompilerParams(
            dimension_semantics=("arbitrary",),
            vmem_limit_bytes=100 * 1024 * 1024),
    )(x_p, wf_p, bias_p, whh_p, w2_p, b2_p)

    return jnp.transpose(out_p[:L, :B, :O], (1, 0, 2))


def kernel(x, w1, b1, wih, whh, bih, bhh, w2, b2):
    params = {
        "w1": w1, "b1": b1,
        "wih": wih, "whh": whh, "bih": bih, "bhh": bhh,
        "w2": w2, "b2": b2,
    }
    return _forward(x, params, hidden_dim=512, output_dim=256)


# speedup vs baseline: 1.2749x; 1.1289x over previous
import functools

import jax
import jax.numpy as jnp
from jax.experimental import pallas as pl
from jax.experimental.pallas import tpu as pltpu


def _lstm_body(
    x_ref,
    wf_ref,
    bias_ref,
    whh_ref,
    w2_ref,
    b2_ref,
    out_ref,
    gx_scr,
    h_scr,
    hc_scr,
    c_scr,
):
    Bb, TL, Dinp = x_ref.shape
    Hp = h_scr.shape[-1]
    Op = out_ref.shape[-1]

    t_id = pl.program_id(0)

    @pl.when(t_id == 0)
    def _():
        hc_scr[...] = jnp.zeros((Bb, Hp), dtype=hc_scr.dtype)
        c_scr[...] = jnp.zeros((Bb, Hp), dtype=c_scr.dtype)

    x2 = x_ref[...].astype(jnp.bfloat16).reshape(Bb * TL, Dinp)
    gx = jnp.dot(x2, wf_ref[...], preferred_element_type=jnp.float32) \
        + bias_ref[...]
    gx_scr[...] = gx.reshape(Bb, TL, 4 * Hp).astype(gx_scr.dtype).swapaxes(0, 1)

    whh = whh_ref[...]

    def step(t, carry):
        h_prev, c_prev = carry
        gates = gx_scr[t].astype(jnp.float32) + jnp.dot(
            h_prev, whh, preferred_element_type=jnp.float32)
        ti = jnp.tanh(gates[:, 0 * Hp:1 * Hp])
        tf = jnp.tanh(gates[:, 1 * Hp:2 * Hp])
        tg = jnp.tanh(gates[:, 2 * Hp:3 * Hp])
        to = jnp.tanh(gates[:, 3 * Hp:4 * Hp])
        i_g = 0.5 + 0.5 * ti
        f_g = 0.5 + 0.5 * tf
        o_g = 0.5 + 0.5 * to
        c_new = f_g * c_prev + i_g * tg
        h_new = (o_g * jnp.tanh(c_new)).astype(h_scr.dtype)
        h_scr[t] = h_new
        return h_new, c_new

    h0 = hc_scr[...]
    c0 = c_scr[...]
    hN, cN = jax.lax.fori_loop(0, TL, step, (h0, c0), unroll=4)
    hc_scr[...] = hN
    c_scr[...] = cN

    h2 = h_scr[...].reshape(TL * Bb, Hp)
    y = jnp.dot(h2, w2_ref[...], preferred_element_type=jnp.float32) \
        + b2_ref[...]
    out_ref[...] = y.reshape(TL, Bb, Op).swapaxes(0, 1)


def _round_up(n, m):
    return ((n + m - 1) // m) * m


def _pad_to(a, shape):
    if tuple(a.shape) == tuple(shape):
        return a
    return jnp.pad(a, [(0, s - d) for d, s in zip(a.shape, shape)])


def _gate_scale_cols(a, H):
    scale = jnp.concatenate([
        jnp.full((H,), 0.5, a.dtype), jnp.full((H,), 0.5, a.dtype),
        jnp.ones((H,), a.dtype), jnp.full((H,), 0.5, a.dtype)
    ])
    return a * scale


def _pad_gate_cols(a, H, Hp):
    if H == Hp:
        return a
    parts = []
    for k in range(4):
        blk = a[..., k * H:(k + 1) * H]
        parts.append(jnp.pad(blk, [(0, 0)] * (a.ndim - 1) + [(0, Hp - H)]))
    return jnp.concatenate(parts, axis=-1)


@functools.partial(jax.jit, static_argnames=("hidden_dim", "output_dim"))
def _forward(x, params, hidden_dim, output_dim):
    B, L, Din = x.shape
    H, O = hidden_dim, output_dim
    f32, bf16 = jnp.float32, jnp.bfloat16

    Bp = _round_up(B, 8)
    Dinp = _round_up(Din, 128)
    Hp = _round_up(H, 128)
    Op = _round_up(O, 128)

    TL = min(64, L)
    Lp = _round_up(L, TL)

    wf = _gate_scale_cols(params["w1"] @ params["wih"], H)
    bias = _gate_scale_cols(
        params["b1"] @ params["wih"] + params["bih"] + params["bhh"], H)
    whh = _gate_scale_cols(params["whh"], H)

    wf_p = _pad_gate_cols(_pad_to(wf, (Dinp, 4 * H)), H, Hp).astype(bf16)
    bias_p = _pad_gate_cols(bias, H, Hp).astype(f32)
    whh_p = _pad_gate_cols(_pad_to(whh, (Hp, 4 * H)), H, Hp).astype(bf16)
    w2_p = _pad_to(params["w2"], (Hp, Op)).astype(bf16)
    b2_p = _pad_to(params["b2"], (1, Op)).astype(f32)

    x_p = _pad_to(x, (Bp, Lp, Dinp))

    out_p = pl.pallas_call(
        _lstm_body,
        out_shape=jax.ShapeDtypeStruct((Bp, Lp, Op), jnp.float32),
        grid_spec=pltpu.PrefetchScalarGridSpec(
            num_scalar_prefetch=0,
            grid=(Lp // TL,),
            in_specs=[
                pl.BlockSpec((Bp, TL, Dinp), lambda l: (0, l, 0)),
                pl.BlockSpec((Dinp, 4 * Hp), lambda l: (0, 0),
                             pipeline_mode=pl.Buffered(1)),
                pl.BlockSpec((1, 4 * Hp), lambda l: (0, 0),
                             pipeline_mode=pl.Buffered(1)),
                pl.BlockSpec((Hp, 4 * Hp), lambda l: (0, 0),
                             pipeline_mode=pl.Buffered(1)),
                pl.BlockSpec((Hp, Op), lambda l: (0, 0),
                             pipeline_mode=pl.Buffered(1)),
                pl.BlockSpec((1, Op), lambda l: (0, 0),
                             pipeline_mode=pl.Buffered(1)),
            ],
            out_specs=pl.BlockSpec((Bp, TL, Op), lambda l: (0, l, 0)),
            scratch_shapes=[
                pltpu.VMEM((TL, Bp, 4 * Hp), jnp.bfloat16),
                pltpu.VMEM((TL, Bp, Hp), jnp.bfloat16),
                pltpu.VMEM((Bp, Hp), jnp.bfloat16),
                pltpu.VMEM((Bp, Hp), jnp.float32),
            ],
        ),
        compiler_params=pltpu.CompilerParams(
            dimension_semantics=("arbitrary",),
            vmem_limit_bytes=100 * 1024 * 1024),
    )(x_p, wf_p, bias_p, whh_p, w2_p, b2_p)

    return out_p[:B, :L, :O]


def kernel(x, w1, b1, wih, whh, bih, bhh, w2, b2):
    params = {
        "w1": w1, "b1": b1,
        "wih": wih, "whh": whh, "bih": bih, "bhh": bhh,
        "w2": w2, "b2": b2,
    }
    return _forward(x, params, hidden_dim=512, output_dim=256)


# speedup vs baseline: 1.2774x; 1.0020x over previous
import functools

import jax
import jax.numpy as jnp
from jax.experimental import pallas as pl
from jax.experimental.pallas import tpu as pltpu


def _lstm_body(
    x_ref,
    wf_ref,
    bias_ref,
    whh_ref,
    w2_ref,
    b2_ref,
    out_ref,
    gx_scr,
    h_scr,
    hc_scr,
    c_scr,
):
    Bb, TL, Dinp = x_ref.shape
    Hp = h_scr.shape[-1]
    Op = out_ref.shape[-1]

    t_id = pl.program_id(0)

    @pl.when(t_id == 0)
    def _():
        hc_scr[...] = jnp.zeros((Bb, Hp), dtype=hc_scr.dtype)
        c_scr[...] = jnp.zeros((Bb, Hp), dtype=c_scr.dtype)

    x2 = x_ref[...].astype(jnp.bfloat16).reshape(Bb * TL, Dinp)
    gx = jnp.dot(x2, wf_ref[...], preferred_element_type=jnp.float32) \
        + bias_ref[...]
    gx_scr[...] = gx.reshape(Bb, TL, 4 * Hp).astype(gx_scr.dtype).swapaxes(0, 1)

    whh = whh_ref[...]

    def step(t, carry):
        h_prev, c_prev = carry
        gates = gx_scr[t].astype(jnp.float32) + jnp.dot(
            h_prev, whh, preferred_element_type=jnp.float32)
        gb = gates.astype(jnp.bfloat16)
        half = jnp.bfloat16(0.5)
        ti = jnp.tanh(gb[:, 0 * Hp:1 * Hp])
        tf = jnp.tanh(gb[:, 1 * Hp:2 * Hp])
        tg = jnp.tanh(gb[:, 2 * Hp:3 * Hp])
        to = jnp.tanh(gb[:, 3 * Hp:4 * Hp])
        i_g = half + half * ti
        f_g = half + half * tf
        o_g = half + half * to
        c_new = f_g.astype(jnp.float32) * c_prev \
            + (i_g * tg).astype(jnp.float32)
        tc = jnp.tanh(c_new.astype(jnp.bfloat16))
        h_new = o_g * tc
        h_scr[t] = h_new
        return h_new, c_new

    h0 = hc_scr[...]
    c0 = c_scr[...]
    hN, cN = jax.lax.fori_loop(0, TL, step, (h0, c0), unroll=4)
    hc_scr[...] = hN
    c_scr[...] = cN

    h2 = h_scr[...].reshape(TL * Bb, Hp)
    y = jnp.dot(h2, w2_ref[...], preferred_element_type=jnp.float32) \
        + b2_ref[...]
    out_ref[...] = y.reshape(TL, Bb, Op).swapaxes(0, 1)


def _round_up(n, m):
    return ((n + m - 1) // m) * m


def _pad_to(a, shape):
    if tuple(a.shape) == tuple(shape):
        return a
    return jnp.pad(a, [(0, s - d) for d, s in zip(a.shape, shape)])


def _gate_scale_cols(a, H):
    scale = jnp.concatenate([
        jnp.full((H,), 0.5, a.dtype), jnp.full((H,), 0.5, a.dtype),
        jnp.ones((H,), a.dtype), jnp.full((H,), 0.5, a.dtype)
    ])
    return a * scale


def _pad_gate_cols(a, H, Hp):
    if H == Hp:
        return a
    parts = []
    for k in range(4):
        blk = a[..., k * H:(k + 1) * H]
        parts.append(jnp.pad(blk, [(0, 0)] * (a.ndim - 1) + [(0, Hp - H)]))
    return jnp.concatenate(parts, axis=-1)


@functools.partial(jax.jit, static_argnames=("hidden_dim", "output_dim"))
def _forward(x, params, hidden_dim, output_dim):
    B, L, Din = x.shape
    H, O = hidden_dim, output_dim
    f32, bf16 = jnp.float32, jnp.bfloat16

    Bp = _round_up(B, 8)
    Dinp = _round_up(Din, 128)
    Hp = _round_up(H, 128)
    Op = _round_up(O, 128)

    TL = min(64, L)
    Lp = _round_up(L, TL)

    wf = _gate_scale_cols(params["w1"] @ params["wih"], H)
    bias = _gate_scale_cols(
        params["b1"] @ params["wih"] + params["bih"] + params["bhh"], H)
    whh = _gate_scale_cols(params["whh"], H)

    wf_p = _pad_gate_cols(_pad_to(wf, (Dinp, 4 * H)), H, Hp).astype(bf16)
    bias_p = _pad_gate_cols(bias, H, Hp).astype(f32)
    whh_p = _pad_gate_cols(_pad_to(whh, (Hp, 4 * H)), H, Hp).astype(bf16)
    w2_p = _pad_to(params["w2"], (Hp, Op)).astype(bf16)
    b2_p = _pad_to(params["b2"], (1, Op)).astype(f32)

    x_p = _pad_to(x, (Bp, Lp, Dinp))

    out_p = pl.pallas_call(
        _lstm_body,
        out_shape=jax.ShapeDtypeStruct((Bp, Lp, Op), jnp.float32),
        grid_spec=pltpu.PrefetchScalarGridSpec(
            num_scalar_prefetch=0,
            grid=(Lp // TL,),
            in_specs=[
                pl.BlockSpec((Bp, TL, Dinp), lambda l: (0, l, 0)),
                pl.BlockSpec((Dinp, 4 * Hp), lambda l: (0, 0),
                             pipeline_mode=pl.Buffered(1)),
                pl.BlockSpec((1, 4 * Hp), lambda l: (0, 0),
                             pipeline_mode=pl.Buffered(1)),
                pl.BlockSpec((Hp, 4 * Hp), lambda l: (0, 0),
                             pipeline_mode=pl.Buffered(1)),
                pl.BlockSpec((Hp, Op), lambda l: (0, 0),
                             pipeline_mode=pl.Buffered(1)),
                pl.BlockSpec((1, Op), lambda l: (0, 0),
                             pipeline_mode=pl.Buffered(1)),
            ],
            out_specs=pl.BlockSpec((Bp, TL, Op), lambda l: (0, l, 0)),
            scratch_shapes=[
                pltpu.VMEM((TL, Bp, 4 * Hp), jnp.bfloat16),
                pltpu.VMEM((TL, Bp, Hp), jnp.bfloat16),
                pltpu.VMEM((Bp, Hp), jnp.bfloat16),
                pltpu.VMEM((Bp, Hp), jnp.float32),
            ],
        ),
        compiler_params=pltpu.CompilerParams(
            dimension_semantics=("arbitrary",),
            vmem_limit_bytes=100 * 1024 * 1024),
    )(x_p, wf_p, bias_p, whh_p, w2_p, b2_p)

    return out_p[:B, :L, :O]


def kernel(x, w1, b1, wih, whh, bih, bhh, w2, b2):
    params = {
        "w1": w1, "b1": b1,
        "wih": wih, "whh": whh, "bih": bih, "bhh": bhh,
        "w2": w2, "b2": b2,
    }
    return _forward(x, params, hidden_dim=512, output_dim=256)
